# baseline (speedup 1.0000x reference)
import functools

import jax
import jax.numpy as jnp
from jax import lax
from jax.experimental import pallas as pl
from jax.experimental.pallas import tpu as pltpu
from jax.experimental.pallas import tpu_sc as plsc

N_NODES = 10000
N_PAD = 10240
DUMMY = 10000
D = 128
N_EDGES = 320000
NW = 32
CHUNK = 128
CHUNKS_PW = 80
E_PW = CHUNK * CHUNKS_PW
E_PAD = NW * E_PW
CPW0 = 100
CPW1 = 60
CPW_MAX = 100
MAXE = CPW_MAX * CHUNK
ROWS_PT = N_PAD // 16
NG = 64
R = 1024
GRID_R = N_PAD // R

_sc_mesh = plsc.VectorSubcoreMesh(core_axis_name="c", subcore_axis_name="s")
_sc_params = pltpu.CompilerParams(needs_layout_passes=False)


@functools.partial(
    pl.kernel,
    out_type=jax.ShapeDtypeStruct((NW, N_PAD), jnp.float32),
    mesh=_sc_mesh,
    scratch_types=[
        pltpu.VMEM((E_PW,), jnp.int32),
        pltpu.VMEM((N_PAD,), jnp.float32),
    ],
    compiler_params=_sc_params,
)
def _sc_degree(dst_hbm, out_hbm, dst_v, deg_v):
    c = lax.axis_index("c")
    s = lax.axis_index("s")
    wid = s * 2 + c
    pltpu.sync_copy(dst_hbm.at[wid], dst_v)

    ones = jnp.ones((16,), jnp.float32)
    zeros = jnp.zeros((16,), jnp.float32)

    def zero_body(r, _):
        deg_v[pl.ds(r * 16, 16)] = zeros
        return 0
    lax.fori_loop(0, N_PAD // 16, zero_body, 0)

    def acc_body(r, _):
        idx = dst_v[pl.ds(r * 16, 16)]
        plsc.addupdate_scatter(deg_v, [idx], ones)
        return 0
    lax.fori_loop(0, E_PW // 16, acc_body, 0)

    pltpu.sync_copy(deg_v, out_hbm.at[wid])


NB = 2
ZROWS = 80


@functools.partial(
    pl.kernel,
    out_type=jax.ShapeDtypeStruct((2, N_PAD, D), jnp.float32),
    mesh=_sc_mesh,
    scratch_types=[
        pltpu.VMEM((MAXE,), jnp.int32),
        pltpu.VMEM((NB, CHUNK), jnp.int32),
        pltpu.VMEM((NB, CHUNK, D), jnp.float32),
        pltpu.VMEM_SHARED((N_PAD, D), jnp.float32),
        pltpu.SemaphoreType.DMA,
        pltpu.SemaphoreType.DMA,
    ],
    compiler_params=_sc_params,
)
def _sc_aggregate(y_hbm, src_hbm, dst_hbm, out_hbm, src_v, dst_v, rows_v,
                  acc, sem0, sem1):
    c = lax.axis_index("c")
    s = lax.axis_index("s")
    wid = s * 2 + c
    sems = [sem0, sem1]
    nch = jnp.where(c == 0, CPW0, CPW1)
    pltpu.sync_copy(src_hbm.at[wid], src_v)

    def gdesc(j, b):
        rows = pltpu.make_async_copy(
            y_hbm.at[src_v.at[pl.ds(j * CHUNK, CHUNK)]], rows_v.at[b],
            sems[b])
        ids = pltpu.make_async_copy(dst_hbm.at[wid].at[j], dst_v.at[b],
                                    sems[b])
        return rows, ids

    zeros = jnp.zeros((16,), jnp.float32)

    def zrow(r, _):
        def zcol(cc, _):
            rows_v[0, r, pl.ds(cc * 16, 16)] = zeros
            return 0
        return lax.fori_loop(0, 8, zcol, 0)
    lax.fori_loop(0, CHUNK, zrow, 0)
    for k in range(ROWS_PT // CHUNK):
        pltpu.sync_copy(rows_v.at[0],
                        acc.at[pl.ds(s * ROWS_PT + k * CHUNK, CHUNK)])

    for b in range(NB):
        for d_ in gdesc(b, b):
            d_.start()
    plsc.subcore_barrier()

    @pl.loop(0, nch, step=NB)
    def _(i):
        for b in range(NB):
            j = i + b
            for d_ in gdesc(j, b):
                d_.wait()
            pltpu.sync_copy(rows_v.at[b], acc.at[dst_v.at[b]], add=True)

            @pl.when(j + NB < nch)
            def _():
                for d_ in gdesc(j + NB, b):
                    d_.start()

    plsc.subcore_barrier()
    pltpu.sync_copy(acc.at[pl.ds(s * ROWS_PT, ROWS_PT)],
                    out_hbm.at[c].at[pl.ds(s * ROWS_PT, ROWS_PT)])


def _dinv_body(parts_ref, dinv_ref, dinv2_ref):
    row = lax.broadcasted_iota(jnp.int32, (80, 128), 0)
    col = lax.broadcasted_iota(jnp.int32, (80, 128), 1)
    n = row * 128 + col
    deg = jnp.sum(parts_ref[...], axis=0) + 1.0
    r = lax.rsqrt(deg)
    dinv = jnp.where(n < N_NODES, r, 0.0)
    dinv_ref[...] = dinv
    dinv2_ref[...] = dinv * dinv


def _layer1_body(x_ref, w_ref, dinv_ref, xw_ref, y_ref):
    xw = jnp.dot(x_ref[...], w_ref[...], preferred_element_type=jnp.float32)
    xw_ref[...] = xw
    y_ref[...] = dinv_ref[...] * xw


def _layer_body(xwp_ref, a0_ref, a1_ref, dinv_ref, dinv2_ref, b_ref, w_ref,
                xw_ref, y_ref):
    h = jax.nn.relu(dinv2_ref[...] * xwp_ref[...]
                    + dinv_ref[...] * (a0_ref[...] + a1_ref[...])
                    + b_ref[...])
    xw = jnp.dot(h, w_ref[...], preferred_element_type=jnp.float32)
    xw_ref[...] = xw
    y_ref[...] = dinv_ref[...] * xw


def _pool_body(xwp_ref, a0_ref, a1_ref, dinv_ref, dinv2_ref, b_ref,
               batch_ref, sums_ref, cnts_ref):
    i = pl.program_id(0)
    h = jax.nn.relu(dinv2_ref[...] * xwp_ref[...]
                    + dinv_ref[...] * (a0_ref[...] + a1_ref[...])
                    + b_ref[...])
    gids = lax.broadcasted_iota(jnp.int32, (1, NG), 1)
    mask = (batch_ref[...] == gids).astype(jnp.float32)
    part = lax.dot_general(mask, h, (((0,), (0,)), ((), ())),
                           preferred_element_type=jnp.float32)
    cpart = lax.dot_general(mask, jnp.ones_like(h), (((0,), (0,)), ((), ())),
                            preferred_element_type=jnp.float32)

    @pl.when(i == 0)
    def _():
        sums_ref[...] = part
        cnts_ref[...] = cpart

    @pl.when(i > 0)
    def _():
        sums_ref[...] += part
        cnts_ref[...] += cpart


def _head_body(sums_ref, cnts_ref, wh1_ref, bh1_ref, wh2_ref, bh2_ref,
               out_ref):
    g = sums_ref[...] / jnp.clip(cnts_ref[...], 1.0, None)
    z = jax.nn.relu(
        jnp.dot(g, wh1_ref[...], preferred_element_type=jnp.float32)
        + bh1_ref[...])
    out_ref[...] = (jnp.dot(z, wh2_ref[...], preferred_element_type=jnp.float32)
                    + bh2_ref[...])


def _row_spec(shape):
    return pl.BlockSpec(shape, lambda i: (i, 0))


def _full_spec(shape):
    return pl.BlockSpec(shape, lambda i: (0, 0))


def kernel(x, edge_index, edge_attr, batch, W1, b1, W2, b2, W3, b3,
           Wh1, bh1, Wh2, bh2):
    f32 = jnp.float32
    x_pad = jnp.pad(x.astype(f32), ((0, N_PAD - N_NODES), (0, 0)))
    src = edge_index[0].astype(jnp.int32)
    dst = edge_index[1].astype(jnp.int32)
    pad_ids = jnp.full((E_PAD - N_EDGES,), DUMMY, jnp.int32)
    src_pad = jnp.concatenate([src, pad_ids])
    dst_pad = jnp.concatenate([dst, pad_ids])
    dst_flat = dst_pad.reshape(NW, E_PW)

    counts = [CPW0 * CHUNK if w % 2 == 0 else CPW1 * CHUNK
              for w in range(NW)]
    offs = [0]
    for n_ in counts:
        offs.append(offs[-1] + n_)
    fill = jnp.full((MAXE,), DUMMY, jnp.int32)

    def _worker_rows(flat):
        out = []
        for w in range(NW):
            sl = flat[offs[w]:offs[w + 1]]
            if counts[w] < MAXE:
                sl = jnp.concatenate([sl, fill[:MAXE - counts[w]]])
            out.append(sl)
        return jnp.stack(out)

    src_w = _worker_rows(src_pad)
    dst_w = _worker_rows(dst_pad).reshape(NW, CPW_MAX, CHUNK)
    batch_pad = jnp.pad(batch.astype(jnp.int32), (0, N_PAD - N_NODES),
                        constant_values=NG).reshape(N_PAD, 1)

    deg_parts = _sc_degree(dst_flat).reshape(NW, 80, 128)
    dinv2d, dinv2_2d = pl.pallas_call(
        _dinv_body,
        grid=(1,),
        in_specs=[pl.BlockSpec((NW, 80, 128), lambda i: (0, 0, 0))],
        out_specs=[_full_spec((80, 128)), _full_spec((80, 128))],
        out_shape=[jax.ShapeDtypeStruct((80, 128), f32)] * 2,
    )(deg_parts)
    dinv = dinv2d.reshape(N_PAD, 1)
    dinv2 = dinv2_2d.reshape(N_PAD, 1)

    xw1, y1 = pl.pallas_call(
        _layer1_body,
        grid=(GRID_R,),
        in_specs=[_row_spec((R, D)), _full_spec((D, D)), _row_spec((R, 1))],
        out_specs=[_row_spec((R, D)), _row_spec((R, D))],
        out_shape=[jax.ShapeDtypeStruct((N_PAD, D), f32)] * 2,
    )(x_pad, W1, dinv)

    def mid_layer(xw_prev, y_prev, b_prev, W):
        accp = _sc_aggregate(y_prev, src_w, dst_w)
        return pl.pallas_call(
            _layer_body,
            grid=(GRID_R,),
            in_specs=[_row_spec((R, D)), _row_spec((R, D)), _row_spec((R, D)),
                      _row_spec((R, 1)), _row_spec((R, 1)),
                      _full_spec((1, D)), _full_spec((D, D))],
            out_specs=[_row_spec((R, D)), _row_spec((R, D))],
            out_shape=[jax.ShapeDtypeStruct((N_PAD, D), f32)] * 2,
        )(xw_prev, accp[0], accp[1], dinv, dinv2, b_prev.reshape(1, D), W)

    xw2, y2 = mid_layer(xw1, y1, b1, W2)
    xw3, y3 = mid_layer(xw2, y2, b2, W3)

    accp3 = _sc_aggregate(y3, src_w, dst_w)
    sums, cnts = pl.pallas_call(
        _pool_body,
        grid=(GRID_R,),
        in_specs=[_row_spec((R, D)), _row_spec((R, D)), _row_spec((R, D)),
                  _row_spec((R, 1)), _row_spec((R, 1)), _full_spec((1, D)),
                  _row_spec((R, 1))],
        out_specs=[_full_spec((NG, D)), _full_spec((NG, D))],
        out_shape=[jax.ShapeDtypeStruct((NG, D), f32)] * 2,
    )(xw3, accp3[0], accp3[1], dinv, dinv2, b3.reshape(1, D), batch_pad)

    out = pl.pallas_call(
        _head_body,
        grid=(1,),
        in_specs=[_full_spec((NG, D)), _full_spec((NG, D)),
                  _full_spec((D, NG)), _full_spec((1, NG)),
                  _full_spec((NG, 1)), _full_spec((1, 1))],
        out_specs=_full_spec((NG, 1)),
        out_shape=jax.ShapeDtypeStruct((NG, 1), f32),
    )(sums, cnts, Wh1, bh1.reshape(1, NG), Wh2, bh2.reshape(1, 1))
    return out


# speedup vs baseline: 1.3492x; 1.3492x over previous
import functools

from functools import partial

import jax
import jax.numpy as jnp
from jax import lax
from jax.experimental import pallas as pl
from jax.experimental.pallas import tpu as pltpu
from jax.experimental.pallas import tpu_sc as plsc

N_NODES = 10000
N_PAD = 10240
DUMMY = 10000
D = 128
N_EDGES = 320000
NW = 32
CHUNK = 128
CHUNKS_PW = 80
E_PW = CHUNK * CHUNKS_PW
E_PAD = NW * E_PW
ROWS_PT = N_PAD // 16
NG = 64
R = 1024
GRID_R = N_PAD // R

_sc_mesh = plsc.VectorSubcoreMesh(core_axis_name="c", subcore_axis_name="s")
_sc_params = pltpu.CompilerParams(needs_layout_passes=False)


@functools.partial(
    pl.kernel,
    out_type=jax.ShapeDtypeStruct((NW, N_PAD), jnp.float32),
    mesh=_sc_mesh,
    scratch_types=[
        pltpu.VMEM((E_PW,), jnp.int32),
        pltpu.VMEM((N_PAD,), jnp.float32),
    ],
    compiler_params=_sc_params,
)
def _sc_degree(dst_hbm, out_hbm, dst_v, deg_v):
    c = lax.axis_index("c")
    s = lax.axis_index("s")
    wid = s * 2 + c
    pltpu.sync_copy(dst_hbm.at[wid], dst_v)

    ones = jnp.ones((16,), jnp.float32)
    zeros = jnp.zeros((16,), jnp.float32)

    def zero_body(r, _):
        deg_v[pl.ds(r * 16, 16)] = zeros
        return 0
    lax.fori_loop(0, N_PAD // 16, zero_body, 0)

    def acc_body(r, _):
        idx = dst_v[pl.ds(r * 16, 16)]
        plsc.addupdate_scatter(deg_v, [idx], ones)
        return 0
    lax.fori_loop(0, E_PW // 16, acc_body, 0)

    pltpu.sync_copy(deg_v, out_hbm.at[wid])


NB = 2
ZROWS = 80


@functools.partial(
    pl.kernel,
    out_type=jax.ShapeDtypeStruct((2, N_PAD, D), jnp.float32),
    mesh=_sc_mesh,
    scratch_types=[
        pltpu.VMEM((E_PW,), jnp.int32),
        pltpu.VMEM((NB, CHUNK), jnp.int32),
        pltpu.VMEM((NB, CHUNK, D), jnp.float32),
        pltpu.VMEM_SHARED((N_PAD, D), jnp.float32),
        pltpu.SemaphoreType.DMA,
        pltpu.SemaphoreType.DMA,
    ],
    compiler_params=_sc_params,
)
def _sc_aggregate(y_hbm, src_hbm, dst_hbm, out_hbm, src_v, dst_v, rows_v,
                  acc, sem0, sem1):
    c = lax.axis_index("c")
    s = lax.axis_index("s")
    wid = s * 2 + c
    sems = [sem0, sem1]
    pltpu.sync_copy(src_hbm.at[wid], src_v)

    def gdesc(j, b):
        rows = pltpu.make_async_copy(
            y_hbm.at[src_v.at[pl.ds(j * CHUNK, CHUNK)]], rows_v.at[b],
            sems[b])
        ids = pltpu.make_async_copy(dst_hbm.at[wid].at[j], dst_v.at[b],
                                    sems[b])
        return rows, ids

    zeros = jnp.zeros((16,), jnp.float32)

    def zrow(r, _):
        def zcol(cc, _):
            rows_v[0, r, pl.ds(cc * 16, 16)] = zeros
            return 0
        return lax.fori_loop(0, 8, zcol, 0)
    lax.fori_loop(0, CHUNK, zrow, 0)
    for k in range(ROWS_PT // CHUNK):
        pltpu.sync_copy(rows_v.at[0],
                        acc.at[pl.ds(s * ROWS_PT + k * CHUNK, CHUNK)])

    for b in range(NB):
        for d_ in gdesc(b, b):
            d_.start()
    plsc.subcore_barrier()

    @pl.loop(0, CHUNKS_PW, step=NB)
    def _(i):
        for b in range(NB):
            j = i + b
            for d_ in gdesc(j, b):
                d_.wait()
            pltpu.sync_copy(rows_v.at[b], acc.at[dst_v.at[b]], add=True)

            @pl.when(j + NB < CHUNKS_PW)
            def _():
                for d_ in gdesc(j + NB, b):
                    d_.start()

    plsc.subcore_barrier()
    pltpu.sync_copy(acc.at[pl.ds(s * ROWS_PT, ROWS_PT)],
                    out_hbm.at[c].at[pl.ds(s * ROWS_PT, ROWS_PT)])


def _dinv_body(parts_ref, dinv_ref, dinv2_ref):
    row = lax.broadcasted_iota(jnp.int32, (80, 128), 0)
    col = lax.broadcasted_iota(jnp.int32, (80, 128), 1)
    n = row * 128 + col
    deg = jnp.sum(parts_ref[...], axis=0) + 1.0
    r = lax.rsqrt(deg)
    dinv = jnp.where(n < N_NODES, r, 0.0)
    dinv_ref[...] = dinv
    dinv2_ref[...] = dinv * dinv


def _layer1_body(x_ref, w_ref, dinv_ref, xw_ref, y_ref):
    xw = jnp.dot(x_ref[...], w_ref[...], preferred_element_type=jnp.float32)
    xw_ref[...] = xw
    y_ref[...] = dinv_ref[...] * xw


def _layer_body(xwp_ref, a0_ref, a1_ref, dinv_ref, dinv2_ref, b_ref, w_ref,
                xw_ref, y_ref):
    h = jax.nn.relu(dinv2_ref[...] * xwp_ref[...]
                    + dinv_ref[...] * (a0_ref[...] + a1_ref[...])
                    + b_ref[...])
    xw = jnp.dot(h, w_ref[...], preferred_element_type=jnp.float32)
    xw_ref[...] = xw
    y_ref[...] = dinv_ref[...] * xw


def _combine_body(xwp_ref, a0_ref, a1_ref, dinv_ref, dinv2_ref, b_ref, h_ref):
    h_ref[...] = jax.nn.relu(dinv2_ref[...] * xwp_ref[...]
                             + dinv_ref[...] * (a0_ref[...] + a1_ref[...])
                             + b_ref[...])


def _pool_body(xwp_ref, a0_ref, a1_ref, dinv_ref, dinv2_ref, b_ref,
               batch_ref, sums_ref, cnts_ref):
    i = pl.program_id(0)
    h = jax.nn.relu(dinv2_ref[...] * xwp_ref[...]
                    + dinv_ref[...] * (a0_ref[...] + a1_ref[...])
                    + b_ref[...])
    gids = lax.broadcasted_iota(jnp.int32, (1, NG), 1)
    mask = (batch_ref[...] == gids).astype(jnp.float32)
    part = lax.dot_general(mask, h, (((0,), (0,)), ((), ())),
                           preferred_element_type=jnp.float32,
                           precision=lax.Precision.HIGHEST)
    cpart = lax.dot_general(mask, jnp.ones_like(h), (((0,), (0,)), ((), ())),
                            preferred_element_type=jnp.float32,
                            precision=lax.Precision.HIGHEST)

    @pl.when(i == 0)
    def _():
        sums_ref[...] = part
        cnts_ref[...] = cpart

    @pl.when(i > 0)
    def _():
        sums_ref[...] += part
        cnts_ref[...] += cpart


def _head_body(sums_ref, cnts_ref, wh1_ref, bh1_ref, wh2_ref, bh2_ref,
               out_ref):
    g = sums_ref[...] / jnp.clip(cnts_ref[...], 1.0, None)
    z = jax.nn.relu(
        jnp.dot(g, wh1_ref[...], preferred_element_type=jnp.float32)
        + bh1_ref[...])
    out_ref[...] = (jnp.dot(z, wh2_ref[...], preferred_element_type=jnp.float32)
                    + bh2_ref[...])


def _row_spec(shape):
    return pl.BlockSpec(shape, lambda i: (i, 0))


def _full_spec(shape):
    return pl.BlockSpec(shape, lambda i: (0, 0))


def kernel(x, edge_index, edge_attr, batch, W1, b1, W2, b2, W3, b3,
           Wh1, bh1, Wh2, bh2):
    f32 = jnp.float32
    x_pad = jnp.pad(x.astype(f32), ((0, N_PAD - N_NODES), (0, 0)))
    src = edge_index[0].astype(jnp.int32)
    dst = edge_index[1].astype(jnp.int32)
    pad_ids = jnp.full((E_PAD - N_EDGES,), DUMMY, jnp.int32)
    src_w = jnp.concatenate([src, pad_ids]).reshape(NW, E_PW)
    dst_flat = jnp.concatenate([dst, pad_ids]).reshape(NW, E_PW)
    dst_w = dst_flat.reshape(NW, CHUNKS_PW, CHUNK)
    batch_pad = jnp.pad(batch.astype(jnp.int32), (0, N_PAD - N_NODES),
                        constant_values=NG).reshape(N_PAD, 1)

    deg = jnp.sum(_sc_degree(dst_flat), axis=0)[:N_NODES] + 1.0
    dinv_n = jnp.where(deg > 0, deg ** -0.5, 0.0)
    dinv = jnp.pad(dinv_n, (0, N_PAD - N_NODES)).reshape(N_PAD, 1)
    dinv2 = dinv * dinv

    xw1, y1 = pl.pallas_call(
        _layer1_body,
        grid=(GRID_R,),
        in_specs=[_row_spec((R, D)), _full_spec((D, D)), _row_spec((R, 1))],
        out_specs=[_row_spec((R, D)), _row_spec((R, D))],
        out_shape=[jax.ShapeDtypeStruct((N_PAD, D), f32)] * 2,
    )(x_pad, W1, dinv)

    def mid_layer(xw_prev, y_prev, b_prev, W):
        accp = _sc_aggregate(y_prev, src_w, dst_w)
        return pl.pallas_call(
            _layer_body,
            grid=(GRID_R,),
            in_specs=[_row_spec((R, D)), _row_spec((R, D)), _row_spec((R, D)),
                      _row_spec((R, 1)), _row_spec((R, 1)),
                      _full_spec((1, D)), _full_spec((D, D))],
            out_specs=[_row_spec((R, D)), _row_spec((R, D))],
            out_shape=[jax.ShapeDtypeStruct((N_PAD, D), f32)] * 2,
        )(xw_prev, accp[0], accp[1], dinv, dinv2, b_prev.reshape(1, D), W)

    xw2, y2 = mid_layer(xw1, y1, b1, W2)
    xw3, y3 = mid_layer(xw2, y2, b2, W3)

    accp3 = _sc_aggregate(y3, src_w, dst_w)
    sums, cnts = pl.pallas_call(
        _pool_body,
        grid=(GRID_R,),
        in_specs=[_row_spec((R, D)), _row_spec((R, D)), _row_spec((R, D)),
                  _row_spec((R, 1)), _row_spec((R, 1)), _full_spec((1, D)),
                  _row_spec((R, 1))],
        out_specs=[_full_spec((NG, D)), _full_spec((NG, D))],
        out_shape=[jax.ShapeDtypeStruct((NG, D), f32)] * 2,
    )(xw3, accp3[0], accp3[1], dinv, dinv2, b3.reshape(1, D), batch_pad)

    out = pl.pallas_call(
        _head_body,
        grid=(1,),
        in_specs=[_full_spec((NG, D)), _full_spec((NG, D)),
                  _full_spec((D, NG)), _full_spec((1, NG)),
                  _full_spec((NG, 1)), _full_spec((1, 1))],
        out_specs=_full_spec((NG, 1)),
        out_shape=jax.ShapeDtypeStruct((NG, 1), f32),
    )(sums, cnts, Wh1, bh1.reshape(1, NG), Wh2, bh2.reshape(1, 1))
    return out


# speedup vs baseline: 1.3500x; 1.0006x over previous
import functools

import jax
import jax.numpy as jnp
from jax import lax
from jax.experimental import pallas as pl
from jax.experimental.pallas import tpu as pltpu
from jax.experimental.pallas import tpu_sc as plsc

N_NODES = 10000
N_PAD = 10240
DUMMY = 10000
D = 128
N_EDGES = 320000
NW = 32
CHUNK = 128
CHUNKS_PW = 80
E_PW = CHUNK * CHUNKS_PW
E_PAD = NW * E_PW
ROWS_PT = N_PAD // 16
NG = 64
R = 1024
GRID_R = N_PAD // R

_sc_mesh = plsc.VectorSubcoreMesh(core_axis_name="c", subcore_axis_name="s")
_sc_params = pltpu.CompilerParams(needs_layout_passes=False)


@functools.partial(
    pl.kernel,
    out_type=jax.ShapeDtypeStruct((NW, N_PAD), jnp.float32),
    mesh=_sc_mesh,
    scratch_types=[
        pltpu.VMEM((E_PW,), jnp.int32),
        pltpu.VMEM((N_PAD,), jnp.float32),
    ],
    compiler_params=_sc_params,
)
def _sc_degree(dst_hbm, out_hbm, dst_v, deg_v):
    c = lax.axis_index("c")
    s = lax.axis_index("s")
    wid = s * 2 + c
    pltpu.sync_copy(dst_hbm.at[wid], dst_v)

    ones = jnp.ones((16,), jnp.float32)
    zeros = jnp.zeros((16,), jnp.float32)

    def zero_body(r, _):
        deg_v[pl.ds(r * 16, 16)] = zeros
        return 0
    lax.fori_loop(0, N_PAD // 16, zero_body, 0)

    def acc_body(r, _):
        idx = dst_v[pl.ds(r * 16, 16)]
        plsc.addupdate_scatter(deg_v, [idx], ones)
        return 0
    lax.fori_loop(0, E_PW // 16, acc_body, 0)

    pltpu.sync_copy(deg_v, out_hbm.at[wid])


NB = 2


@functools.partial(
    pl.kernel,
    out_type=jax.ShapeDtypeStruct((2, N_PAD, D), jnp.float32),
    mesh=_sc_mesh,
    scratch_types=[
        pltpu.VMEM((E_PW,), jnp.int32),
        pltpu.VMEM((NB, CHUNK), jnp.int32),
        pltpu.VMEM((NB, CHUNK, D), jnp.float32),
        pltpu.VMEM_SHARED((N_PAD, D), jnp.float32),
        pltpu.SemaphoreType.DMA,
        pltpu.SemaphoreType.DMA,
    ],
    compiler_params=_sc_params,
)
def _sc_aggregate(y_hbm, src_hbm, dst_hbm, out_hbm, src_v, dst_v, rows_v,
                  acc, sem0, sem1):
    c = lax.axis_index("c")
    s = lax.axis_index("s")
    wid = s * 2 + c
    sems = [sem0, sem1]
    pltpu.sync_copy(src_hbm.at[wid], src_v)

    def gdesc(j, b):
        rows = pltpu.make_async_copy(
            y_hbm.at[src_v.at[pl.ds(j * CHUNK, CHUNK)]], rows_v.at[b],
            sems[b])
        ids = pltpu.make_async_copy(dst_hbm.at[wid].at[j], dst_v.at[b],
                                    sems[b])
        return rows, ids

    zeros = jnp.zeros((16,), jnp.float32)

    def zrow(r, _):
        def zcol(cc, _):
            rows_v[0, r, pl.ds(cc * 16, 16)] = zeros
            return 0
        return lax.fori_loop(0, 8, zcol, 0)
    lax.fori_loop(0, CHUNK, zrow, 0)
    for k in range(ROWS_PT // CHUNK):
        pltpu.sync_copy(rows_v.at[0],
                        acc.at[pl.ds(s * ROWS_PT + k * CHUNK, CHUNK)])

    for b in range(NB):
        for d_ in gdesc(b, b):
            d_.start()
    plsc.subcore_barrier()

    @pl.loop(0, CHUNKS_PW, step=NB)
    def _(i):
        for b in range(NB):
            j = i + b
            for d_ in gdesc(j, b):
                d_.wait()
            pltpu.sync_copy(rows_v.at[b], acc.at[dst_v.at[b]], add=True)

            @pl.when(j + NB < CHUNKS_PW)
            def _():
                for d_ in gdesc(j + NB, b):
                    d_.start()

    plsc.subcore_barrier()
    pltpu.sync_copy(acc.at[pl.ds(s * ROWS_PT, ROWS_PT)],
                    out_hbm.at[c].at[pl.ds(s * ROWS_PT, ROWS_PT)])


def _layer1_body(x_ref, w_ref, dinv_ref, xw_ref, y_ref):
    xw = jnp.dot(x_ref[...], w_ref[...], preferred_element_type=jnp.float32)
    xw_ref[...] = xw
    y_ref[...] = dinv_ref[...] * xw


def _layer_body(xwp_ref, a0_ref, a1_ref, dinv_ref, dinv2_ref, b_ref, w_ref,
                xw_ref, y_ref):
    h = jax.nn.relu(dinv2_ref[...] * xwp_ref[...]
                    + dinv_ref[...] * (a0_ref[...] + a1_ref[...])
                    + b_ref[...])
    xw = jnp.dot(h, w_ref[...], preferred_element_type=jnp.float32)
    xw_ref[...] = xw
    y_ref[...] = dinv_ref[...] * xw


def _pool_body(xwp_ref, a0_ref, a1_ref, dinv_ref, dinv2_ref, b_ref,
               batch_ref, sums_ref, cnts_ref):
    i = pl.program_id(0)
    h = jax.nn.relu(dinv2_ref[...] * xwp_ref[...]
                    + dinv_ref[...] * (a0_ref[...] + a1_ref[...])
                    + b_ref[...])
    gids = lax.broadcasted_iota(jnp.int32, (1, NG), 1)
    mask = (batch_ref[...] == gids).astype(jnp.float32)
    part = lax.dot_general(mask, h, (((0,), (0,)), ((), ())),
                           preferred_element_type=jnp.float32,
                           precision=lax.Precision.HIGHEST)
    cpart = lax.dot_general(mask, jnp.ones_like(h), (((0,), (0,)), ((), ())),
                            preferred_element_type=jnp.float32,
                            precision=lax.Precision.HIGHEST)

    @pl.when(i == 0)
    def _():
        sums_ref[...] = part
        cnts_ref[...] = cpart

    @pl.when(i > 0)
    def _():
        sums_ref[...] += part
        cnts_ref[...] += cpart


def _head_body(sums_ref, cnts_ref, wh1_ref, bh1_ref, wh2_ref, bh2_ref,
               out_ref):
    g = sums_ref[...] / jnp.clip(cnts_ref[...], 1.0, None)
    z = jax.nn.relu(
        jnp.dot(g, wh1_ref[...], preferred_element_type=jnp.float32)
        + bh1_ref[...])
    out_ref[...] = (jnp.dot(z, wh2_ref[...], preferred_element_type=jnp.float32)
                    + bh2_ref[...])


def _row_spec(shape):
    return pl.BlockSpec(shape, lambda i: (i, 0))


def _full_spec(shape):
    return pl.BlockSpec(shape, lambda i: (0, 0))


def kernel(x, edge_index, edge_attr, batch, W1, b1, W2, b2, W3, b3,
           Wh1, bh1, Wh2, bh2):
    f32 = jnp.float32
    x_pad = jnp.pad(x.astype(f32), ((0, N_PAD - N_NODES), (0, 0)))
    src = edge_index[0].astype(jnp.int32)
    dst = edge_index[1].astype(jnp.int32)
    pad_ids = jnp.full((E_PAD - N_EDGES,), DUMMY, jnp.int32)
    src_w = jnp.concatenate([src, pad_ids]).reshape(NW, E_PW)
    dst_flat = jnp.concatenate([dst, pad_ids]).reshape(NW, E_PW)
    dst_w = dst_flat.reshape(NW, CHUNKS_PW, CHUNK)
    batch_pad = jnp.pad(batch.astype(jnp.int32), (0, N_PAD - N_NODES),
                        constant_values=NG).reshape(N_PAD, 1)

    deg = jnp.sum(_sc_degree(dst_flat), axis=0)[:N_NODES] + 1.0
    dinv_n = jnp.where(deg > 0, deg ** -0.5, 0.0)
    dinv = jnp.pad(dinv_n, (0, N_PAD - N_NODES)).reshape(N_PAD, 1)
    dinv2 = dinv * dinv

    xw1, y1 = pl.pallas_call(
        _layer1_body,
        grid=(GRID_R,),
        in_specs=[_row_spec((R, D)), _full_spec((D, D)), _row_spec((R, 1))],
        out_specs=[_row_spec((R, D)), _row_spec((R, D))],
        out_shape=[jax.ShapeDtypeStruct((N_PAD, D), f32)] * 2,
    )(x_pad, W1, dinv)

    def mid_layer(xw_prev, y_prev, b_prev, W):
        accp = _sc_aggregate(y_prev, src_w, dst_w)
        return pl.pallas_call(
            _layer_body,
            grid=(GRID_R,),
            in_specs=[_row_spec((R, D)), _row_spec((R, D)), _row_spec((R, D)),
                      _row_spec((R, 1)), _row_spec((R, 1)),
                      _full_spec((1, D)), _full_spec((D, D))],
            out_specs=[_row_spec((R, D)), _row_spec((R, D))],
            out_shape=[jax.ShapeDtypeStruct((N_PAD, D), f32)] * 2,
        )(xw_prev, accp[0], accp[1], dinv, dinv2, b_prev.reshape(1, D), W)

    xw2, y2 = mid_layer(xw1, y1, b1, W2)
    xw3, y3 = mid_layer(xw2, y2, b2, W3)

    accp3 = _sc_aggregate(y3, src_w, dst_w)
    sums, cnts = pl.pallas_call(
        _pool_body,
        grid=(GRID_R,),
        in_specs=[_row_spec((R, D)), _row_spec((R, D)), _row_spec((R, D)),
                  _row_spec((R, 1)), _row_spec((R, 1)), _full_spec((1, D)),
                  _row_spec((R, 1))],
        out_specs=[_full_spec((NG, D)), _full_spec((NG, D))],
        out_shape=[jax.ShapeDtypeStruct((NG, D), f32)] * 2,
    )(xw3, accp3[0], accp3[1], dinv, dinv2, b3.reshape(1, D), batch_pad)

    out = pl.pallas_call(
        _head_body,
        grid=(1,),
        in_specs=[_full_spec((NG, D)), _full_spec((NG, D)),
                  _full_spec((D, NG)), _full_spec((1, NG)),
                  _full_spec((NG, 1)), _full_spec((1, 1))],
        out_specs=_full_spec((NG, 1)),
        out_shape=jax.ShapeDtypeStruct((NG, 1), f32),
    )(sums, cnts, Wh1, bh1.reshape(1, NG), Wh2, bh2.reshape(1, 1))
    return out
